# baseline (speedup 1.0000x reference)
import functools

import jax
import jax.numpy as jnp
from jax import lax
from jax.experimental import pallas as pl
from jax.experimental.pallas import tpu as pltpu
from jax.experimental.pallas import tpu_sc as plsc

_NC = 2
_NS = 16
_NW = _NC * _NS
_L = 16


def _fea2_sparsecore(r, edge_index):
    n_nodes = r.shape[0]
    e = edge_index.shape[1]
    chunk = 4096
    n_full = e // chunk
    rem = e - n_full * chunk
    assert rem % 128 == 0 and chunk % _L == 0

    mesh = plsc.VectorSubcoreMesh(
        core_axis_name="c", subcore_axis_name="s",
        num_cores=_NC, num_subcores=_NS)

    @functools.partial(
        pl.kernel,
        mesh=mesh,
        out_type=jax.ShapeDtypeStruct((2, e), jnp.float32),
        compiler_params=pltpu.CompilerParams(needs_layout_passes=False),
        scratch_types=[
            pltpu.VMEM((n_nodes,), jnp.float32),
            pltpu.VMEM((2, chunk), jnp.int32),
            pltpu.VMEM((2, chunk), jnp.float32),
        ],
    )
    def k(r_hbm, ei_hbm, out_hbm, r_v, idx_v, o_v):
        wid = lax.axis_index("s") * _NC + lax.axis_index("c")
        pltpu.sync_copy(r_hbm, r_v)

        def run_chunk(base, n):
            pltpu.sync_copy(ei_hbm.at[:, pl.ds(base, n)], idx_v.at[:, pl.ds(0, n)])

            @plsc.parallel_loop(0, n, step=_L, unroll=8)
            def body(off):
                idx_r = idx_v[0, pl.ds(off, _L)]
                idx_c = idx_v[1, pl.ds(off, _L)]
                o_v[0, pl.ds(off, _L)] = plsc.load_gather(r_v, [idx_r])
                o_v[1, pl.ds(off, _L)] = plsc.load_gather(r_v, [idx_c])

            pltpu.sync_copy(o_v.at[:, pl.ds(0, n)], out_hbm.at[:, pl.ds(base, n)])

        n_mine = (n_full - wid + _NW - 1) // _NW

        def do_chunk(i, _):
            run_chunk((wid + i * _NW) * chunk, chunk)
            return 0

        lax.fori_loop(0, n_mine, do_chunk, 0)

        if rem:
            @pl.when(wid == 0)
            def _():
                run_chunk(n_full * chunk, rem)

    return k(r, edge_index)


def _cos_poly(v):
    two_over_pi = 0.6366197723675814
    magic = 12582912.0
    p1 = 1.5703125
    p2 = 4.837512969970703125e-4
    p3 = 7.54978995489188608e-8
    kf2 = v * two_over_pi + magic
    kf = kf2 - magic
    ki = lax.bitcast_convert_type(kf2, jnp.int32)
    y = ((v - kf * p1) - kf * p2) - kf * p3
    z = y * y
    cosp = 1.0 + z * (-0.5 + z * (4.166664568298827e-2
                                  + z * (-1.388731625493765e-3
                                         + z * 2.443315711809948e-5)))
    sinp = y + y * z * (-1.6666654611e-1
                        + z * (8.3321608736e-3 + z * (-1.9515295891e-4)))
    res = jnp.where((ki & 1) == 1, sinp, cosp)
    sign = ((ki + 1) << 30) & jnp.int32(-2147483648)
    return lax.bitcast_convert_type(
        lax.bitcast_convert_type(res, jnp.int32) ^ sign, jnp.float32)


def _fea1_tensorcore(ht):
    d, e = ht.shape
    block_cols = 64000
    assert e % block_cols == 0

    def body(x_ref, o_ref):
        v = x_ref[...]
        sub = lax.broadcasted_iota(jnp.int32, v.shape, 0)
        o_ref[...] = jnp.where(sub == 0, v, _cos_poly(v))

    return pl.pallas_call(
        body,
        grid=(e // block_cols,),
        in_specs=[pl.BlockSpec((d, block_cols), lambda i: (0, i))],
        out_specs=pl.BlockSpec((d, block_cols), lambda i: (0, i)),
        out_shape=jax.ShapeDtypeStruct((d, e), jnp.float32),
    )(ht)


def kernel(r, h, edge_index):
    fea2 = _fea2_sparsecore(r, edge_index.astype(jnp.int32)).T
    fea1 = _fea1_tensorcore(h.T).T
    return (fea1, fea2)
